# baseline (speedup 1.0000x reference)
import functools

import jax
import jax.numpy as jnp
from jax import lax
from jax.experimental import pallas as pl
from jax.experimental.pallas import tpu as pltpu
from jax.experimental.pallas import tpu_sc as plsc

B = 64
C = 32
U = 5000
UP = 5120
H = 4
D = 128
HIST_BASE = 50000
TOPK = 5
NSLOT = 8
NROWS = B * NSLOT

NC, NS = 2, 16
NW = NC * NS
ROWS_PER_W = NROWS // NW


def _sim_topk_body(ctx_ref, hist_ref, nodes_ref, ids_ref):
    test_bin = (ctx_ref[...] > 0).astype(jnp.float32)
    test0 = test_bin[:, 0:1]
    sim = None
    for h in range(H):
        hs = hist_ref[:, h * UP:(h + 1) * UP]
        hb = (hs > 0).astype(jnp.float32)
        d = jnp.dot(test_bin, hb, preferred_element_type=jnp.float32)
        corr = test0 * (hs[0:1, :] == 0).astype(jnp.float32)
        cm = d + corr
        sim = cm if sim is None else jnp.maximum(sim, cm)
    lane = lax.broadcasted_iota(jnp.int32, (B, UP), 1)
    sim = jnp.where(lane >= U, -1.0, sim)
    slot = lax.broadcasted_iota(jnp.int32, (B, NSLOT), 1)
    ids = jnp.where(slot == 0, nodes_ref[...], 0)
    for k in range(TOPK):
        m = jnp.max(sim, axis=1, keepdims=True)
        cand = jnp.where(sim == m, lane, jnp.int32(1 << 30))
        idx = jnp.min(cand, axis=1, keepdims=True)
        ids = jnp.where(slot == k + 1, idx + HIST_BASE, ids)
        sim = jnp.where(lane == idx, -2.0, sim)
    ids_ref[...] = ids


def _mlp_body(g_ref, w_ref, b_ref, o_ref):
    self_f = g_ref[0:B, :]
    acc = g_ref[B:2 * B, :]
    for k in range(2, TOPK + 1):
        acc = acc + g_ref[k * B:(k + 1) * B, :]
    neigh = acc * (1.0 / TOPK)
    comb = jnp.concatenate([self_f, neigh], axis=1)
    out = jnp.dot(comb, w_ref[...], preferred_element_type=jnp.float32)
    o_ref[...] = jnp.maximum(out + b_ref[...], 0.0)


@functools.cache
def _make_sc_gather():
    mesh = plsc.VectorSubcoreMesh(core_axis_name="c", subcore_axis_name="s")

    @functools.partial(
        pl.kernel, mesh=mesh,
        out_type=jax.ShapeDtypeStruct((NROWS, D), jnp.float32),
        scratch_types=[
            pltpu.VMEM((ROWS_PER_W,), jnp.int32),
            pltpu.VMEM((ROWS_PER_W, D), jnp.float32),
            pltpu.SemaphoreType.DMA,
        ],
    )
    def gather_k(table_hbm, idx_hbm, out_hbm, idx_v, rows_v, sem):
        wid = lax.axis_index("s") * NC + lax.axis_index("c")
        base = wid * ROWS_PER_W
        pltpu.sync_copy(idx_hbm.at[pl.ds(base, ROWS_PER_W)], idx_v)
        pltpu.async_copy(table_hbm.at[idx_v], rows_v, sem).wait()
        pltpu.sync_copy(rows_v, out_hbm.at[pl.ds(base, ROWS_PER_W)])

    return gather_k


def kernel(nodes, context, hist_ctx, features, W1, b1):
    histp = jnp.pad(hist_ctx.transpose(1, 0, 2), ((0, 0), (0, UP - U), (0, 0)))
    histT = histp.reshape(H * UP, C).T
    nodes2d = nodes.reshape(B, 1)

    ids = pl.pallas_call(
        _sim_topk_body,
        out_shape=jax.ShapeDtypeStruct((B, NSLOT), jnp.int32),
    )(context, histT, nodes2d)

    ids_flat = ids.T.reshape(NROWS)
    gathered = _make_sc_gather()(features, ids_flat)

    out = pl.pallas_call(
        _mlp_body,
        out_shape=jax.ShapeDtypeStruct((B, D), jnp.float32),
    )(gathered, W1.T, b1.reshape(1, D))
    return out
